# Initial kernel scaffold; baseline (speedup 1.0000x reference)
import jax
import jax.numpy as jnp
from jax.experimental import pallas as pl


def kernel(x, token_table):
    raise NotImplementedError("write your pallas kernel here")


# speedup vs baseline: 3.0317x; 3.0317x over previous
import functools

import numpy as np
import jax
import jax.numpy as jnp
from jax import lax
from jax.experimental import pallas as pl
from jax.experimental.pallas import tpu as pltpu, tpu_sc as plsc

VOCAB = 100000
D_MODEL = 1024
BATCH = 4
SEQ = 4096

_NC = 2
_NS = 16
_NW = _NC * _NS
_POS_PER_W = SEQ // _NW
_C = 32
_K = _POS_PER_W // _C
_LANES = 16
_VECS = D_MODEL // _LANES


def _pe_table() -> np.ndarray:
    pos = np.arange(SEQ, dtype=np.float32)[:, None]
    two_i = np.arange(0, D_MODEL, 2, dtype=np.float32)
    div = np.power(10000.0, two_i / D_MODEL)
    pe = np.zeros((SEQ, D_MODEL), dtype=np.float32)
    pe[:, 0::2] = np.sin(pos / div)
    pe[:, 1::2] = np.cos(pos / div)
    return pe


_PE = _pe_table()


@functools.partial(
    pl.kernel,
    mesh=plsc.VectorSubcoreMesh(core_axis_name="c", subcore_axis_name="s"),
    out_type=jax.ShapeDtypeStruct((BATCH, SEQ, D_MODEL), jnp.float32),
    scratch_types=[
        pltpu.VMEM((_C,), jnp.int32),
        pltpu.VMEM((_C, D_MODEL), jnp.float32),
        pltpu.VMEM((_C, D_MODEL), jnp.float32),
        pltpu.SemaphoreType.DMA,
    ],
)
def _emb_kernel(table_hbm, x_hbm, pe_hbm, out_hbm, idx_v, pe_v, tok_v, sem):
    wid = lax.axis_index("s") * _NC + lax.axis_index("c")
    pos0 = wid * _POS_PER_W

    def chunk_body(k, _):
        pos = pos0 + k * _C
        pltpu.sync_copy(pe_hbm.at[pl.ds(pos, _C)], pe_v)

        def batch_body(b, _):
            pltpu.sync_copy(x_hbm.at[b, pl.ds(pos, _C)], idx_v)
            pltpu.async_copy(table_hbm.at[idx_v], tok_v, sem).wait()

            def row_body(i, _):
                for j in range(_VECS):
                    sl = pl.ds(j * _LANES, _LANES)
                    tok_v[i, sl] = tok_v[i, sl] + pe_v[i, sl]
                return 0

            lax.fori_loop(0, _C, row_body, 0)
            pltpu.sync_copy(tok_v, out_hbm.at[b, pl.ds(pos, _C)])
            return 0

        lax.fori_loop(0, BATCH, batch_body, 0)
        return 0

    lax.fori_loop(0, _K, chunk_body, 0)


def kernel(x, token_table):
    x = x.astype(jnp.int32)
    pe = jnp.asarray(_PE)
    return _emb_kernel(token_table, x, pe)
